# baseline (speedup 1.0000x reference)
import jax
import jax.numpy as jnp
from jax.experimental import pallas as pl
from jax.experimental.pallas import tpu as pltpu

_CH = 8192
_K = 2
_NCH = 12
_TAIL = 1696
_TSTART = _NCH * _CH


def _oim_body(f_ref, t_ref, l1_ref, l2_ref, l3_ref, l4_ref,
              logits_ref, loss_ref,
              buf_ref, stage_ref, tbuf_ref, tstage_ref,
              sem_in, sem_out, sem_tin, sem_tout):
    lut_refs = (l1_ref, l2_ref, l3_ref, l4_ref)

    def in_copy(b, c):
        return pltpu.make_async_copy(
            lut_refs[b].at[pl.ds(c * _CH, _CH), :],
            buf_ref.at[c % _K, :, pl.ds(b * 128, 128)],
            sem_in.at[b, c % _K])

    def out_copy(c):
        return pltpu.make_async_copy(
            stage_ref.at[c % _K],
            logits_ref.at[:, pl.ds(c * _CH, _CH)],
            sem_out.at[c % _K])

    def tin_copy(b):
        return pltpu.make_async_copy(
            lut_refs[b].at[pl.ds(_TSTART, _TAIL), :],
            tbuf_ref.at[:, pl.ds(b * 128, 128)],
            sem_tin.at[b])

    def tout_copy():
        return pltpu.make_async_copy(
            tstage_ref,
            logits_ref.at[:, pl.ds(_TSTART, _TAIL)],
            sem_tout)

    for c in range(_K):
        for b in range(4):
            in_copy(b, c).start()
    for b in range(4):
        tin_copy(b).start()

    batch = f_ref.shape[0]
    f = f_ref[...].reshape(batch, 4 * 128)
    t = t_ref[...]
    dn = (((1,), (1,)), ((), ()))

    m = jnp.full((batch, 1), -jnp.inf, dtype=jnp.float32)
    s = jnp.zeros((batch, 1), dtype=jnp.float32)
    tl = jnp.zeros((batch, 1), dtype=jnp.float32)
    iota_full = jax.lax.broadcasted_iota(jnp.int32, (batch, _CH), 1)
    iota_tail = iota_full[:, :_TAIL]

    def fold(m, s, tl, acc, base, iota):
        bmax = jnp.max(acc, axis=1, keepdims=True)
        m_new = jnp.maximum(m, bmax)
        p = jnp.exp(acc - m_new)
        s = s * jnp.exp(m - m_new) + jnp.sum(p, axis=1, keepdims=True)
        tl = tl + jnp.sum(jnp.where(iota == t - base, acc, 0.0), axis=1,
                          keepdims=True)
        return m_new, s, tl

    for c in range(_NCH):
        slot = c % _K
        for b in range(4):
            in_copy(b, c).wait()
        acc = jax.lax.dot_general(f, buf_ref[slot], dn,
                                  preferred_element_type=jnp.float32)
        if c >= _K:
            out_copy(c - _K).wait()
        stage_ref[slot] = acc
        out_copy(c).start()
        if c + _K < _NCH:
            for b in range(4):
                in_copy(b, c + _K).start()
        m, s, tl = fold(m, s, tl, acc, c * _CH, iota_full)

    for b in range(4):
        tin_copy(b).wait()
    acc = jax.lax.dot_general(f, tbuf_ref[...], dn,
                              preferred_element_type=jnp.float32)
    tstage_ref[...] = acc
    tout_copy().start()
    m, s, tl = fold(m, s, tl, acc, _TSTART, iota_tail)

    for c in range(_NCH - _K, _NCH):
        out_copy(c).wait()
    tout_copy().wait()

    loss_ref[...] = jnp.mean(m + jnp.log(s) - tl).reshape(1, 1)


def kernel(features, scores, targets, flags, lut_b1, lut_b2, lut_b3,
           lut_b4):
    batch, _, nfeat = features.shape
    nclasses = lut_b1.shape[0]
    t2 = targets.astype(jnp.int32).reshape(batch, 1)

    lut_spec = pl.BlockSpec(memory_space=pl.ANY)
    logits, loss = pl.pallas_call(
        _oim_body,
        in_specs=[
            pl.BlockSpec(memory_space=pltpu.MemorySpace.VMEM),
            pl.BlockSpec(memory_space=pltpu.MemorySpace.VMEM),
            lut_spec, lut_spec, lut_spec, lut_spec,
        ],
        out_specs=[
            pl.BlockSpec(memory_space=pl.ANY),
            pl.BlockSpec(memory_space=pltpu.MemorySpace.VMEM),
        ],
        out_shape=[
            jax.ShapeDtypeStruct((batch, nclasses), jnp.float32),
            jax.ShapeDtypeStruct((1, 1), jnp.float32),
        ],
        scratch_shapes=[
            pltpu.VMEM((_K, _CH, 4 * 128), jnp.float32),
            pltpu.VMEM((_K, batch, _CH), jnp.float32),
            pltpu.VMEM((_TAIL, 4 * 128), jnp.float32),
            pltpu.VMEM((batch, _TAIL), jnp.float32),
            pltpu.SemaphoreType.DMA((4, _K)),
            pltpu.SemaphoreType.DMA((_K,)),
            pltpu.SemaphoreType.DMA((4,)),
            pltpu.SemaphoreType.DMA,
        ],
    )(features, t2, lut_b1, lut_b2, lut_b3, lut_b4)
    return (loss[0, 0], logits)


# speedup vs baseline: 1.0196x; 1.0196x over previous
import jax
import jax.numpy as jnp
from jax.experimental import pallas as pl
from jax.experimental.pallas import tpu as pltpu

_CH = 6144
_K = 3
_NCH = 16
_TAIL = 1696
_TSTART = _NCH * _CH


def _oim_body(f_ref, t_ref, l1_ref, l2_ref, l3_ref, l4_ref,
              logits_ref, loss_ref,
              buf_ref, stage_ref, tbuf_ref, tstage_ref,
              sem_in, sem_out, sem_tin, sem_tout):
    lut_refs = (l1_ref, l2_ref, l3_ref, l4_ref)

    def in_copy(b, c):
        return pltpu.make_async_copy(
            lut_refs[b].at[pl.ds(c * _CH, _CH), :],
            buf_ref.at[c % _K, :, pl.ds(b * 128, 128)],
            sem_in.at[b, c % _K])

    def out_copy(c):
        return pltpu.make_async_copy(
            stage_ref.at[c % _K],
            logits_ref.at[:, pl.ds(c * _CH, _CH)],
            sem_out.at[c % _K])

    def tin_copy(b):
        return pltpu.make_async_copy(
            lut_refs[b].at[pl.ds(_TSTART, _TAIL), :],
            tbuf_ref.at[:, pl.ds(b * 128, 128)],
            sem_tin.at[b])

    def tout_copy():
        return pltpu.make_async_copy(
            tstage_ref,
            logits_ref.at[:, pl.ds(_TSTART, _TAIL)],
            sem_tout)

    for c in range(_K):
        for b in range(4):
            in_copy(b, c).start()
    for b in range(4):
        tin_copy(b).start()

    batch = f_ref.shape[0]
    f = f_ref[...].reshape(batch, 4 * 128)
    t = t_ref[...]
    dn = (((1,), (1,)), ((), ()))

    m = jnp.full((batch, 1), -jnp.inf, dtype=jnp.float32)
    s = jnp.zeros((batch, 1), dtype=jnp.float32)
    tl = jnp.zeros((batch, 1), dtype=jnp.float32)
    iota_full = jax.lax.broadcasted_iota(jnp.int32, (batch, _CH), 1)
    iota_tail = iota_full[:, :_TAIL]

    def fold(m, s, tl, acc, base, iota):
        bmax = jnp.max(acc, axis=1, keepdims=True)
        m_new = jnp.maximum(m, bmax)
        p = jnp.exp(acc - m_new)
        s = s * jnp.exp(m - m_new) + jnp.sum(p, axis=1, keepdims=True)
        tl = tl + jnp.sum(jnp.where(iota == t - base, acc, 0.0), axis=1,
                          keepdims=True)
        return m_new, s, tl

    for c in range(_NCH):
        slot = c % _K
        for b in range(4):
            in_copy(b, c).wait()
        acc = jax.lax.dot_general(f, buf_ref[slot], dn,
                                  preferred_element_type=jnp.float32)
        if c >= _K:
            out_copy(c - _K).wait()
        stage_ref[slot] = acc
        out_copy(c).start()
        if c + _K < _NCH:
            for b in range(4):
                in_copy(b, c + _K).start()
        m, s, tl = fold(m, s, tl, acc, c * _CH, iota_full)

    for b in range(4):
        tin_copy(b).wait()
    acc = jax.lax.dot_general(f, tbuf_ref[...], dn,
                              preferred_element_type=jnp.float32)
    tstage_ref[...] = acc
    tout_copy().start()
    m, s, tl = fold(m, s, tl, acc, _TSTART, iota_tail)

    for c in range(_NCH - _K, _NCH):
        out_copy(c).wait()
    tout_copy().wait()

    loss_ref[...] = jnp.mean(m + jnp.log(s) - tl).reshape(1, 1)


def kernel(features, scores, targets, flags, lut_b1, lut_b2, lut_b3,
           lut_b4):
    batch, _, nfeat = features.shape
    nclasses = lut_b1.shape[0]
    t2 = targets.astype(jnp.int32).reshape(batch, 1)

    lut_spec = pl.BlockSpec(memory_space=pl.ANY)
    logits, loss = pl.pallas_call(
        _oim_body,
        in_specs=[
            pl.BlockSpec(memory_space=pltpu.MemorySpace.VMEM),
            pl.BlockSpec(memory_space=pltpu.MemorySpace.VMEM),
            lut_spec, lut_spec, lut_spec, lut_spec,
        ],
        out_specs=[
            pl.BlockSpec(memory_space=pl.ANY),
            pl.BlockSpec(memory_space=pltpu.MemorySpace.VMEM),
        ],
        out_shape=[
            jax.ShapeDtypeStruct((batch, nclasses), jnp.float32),
            jax.ShapeDtypeStruct((1, 1), jnp.float32),
        ],
        scratch_shapes=[
            pltpu.VMEM((_K, _CH, 4 * 128), jnp.float32),
            pltpu.VMEM((_K, batch, _CH), jnp.float32),
            pltpu.VMEM((_TAIL, 4 * 128), jnp.float32),
            pltpu.VMEM((batch, _TAIL), jnp.float32),
            pltpu.SemaphoreType.DMA((4, _K)),
            pltpu.SemaphoreType.DMA((_K,)),
            pltpu.SemaphoreType.DMA((4,)),
            pltpu.SemaphoreType.DMA,
        ],
    )(features, t2, lut_b1, lut_b2, lut_b3, lut_b4)
    return (loss[0, 0], logits)
